# baseline (speedup 1.0000x reference)
import functools

import jax
import jax.numpy as jnp
from jax import lax
from jax.experimental import pallas as pl
from jax.experimental.pallas import tpu as pltpu
from jax.experimental.pallas import tpu_sc as plsc

NC = 2
NS = 16
L = 16
NW = NC * NS

B = 4096
K = 32
D = 128
BPW = B // NW
DB = D // L
NBUF = 8


def _rsqrt(x):
    i = lax.bitcast_convert_type(x, jnp.int32)
    i = jnp.int32(0x5F3759DF) - jnp.right_shift(i, 1)
    y = lax.bitcast_convert_type(i, jnp.float32)
    for _ in range(3):
        y = y * (jnp.float32(1.5) - jnp.float32(0.5) * x * y * y)
    return y


_mesh = plsc.VectorSubcoreMesh(
    core_axis_name="c", subcore_axis_name="s", num_cores=NC, num_subcores=NS
)


def _make_kernel(interpret=False):
    return functools.partial(
        pl.kernel,
        out_type=jax.ShapeDtypeStruct((B, D), jnp.float32),
        mesh=_mesh,
        compiler_params=pltpu.CompilerParams(
            needs_layout_passes=False, use_tc_tiling_on_sc=False
        ),
        interpret=interpret,
        scratch_types=[
            pltpu.VMEM((BPW,), jnp.int32),
            pltpu.VMEM((K, BPW), jnp.int32),
            pltpu.VMEM((BPW, K), jnp.int32),
            pltpu.VMEM((BPW + L,), jnp.float32),
            tuple(pltpu.VMEM((K, D), jnp.float32) for _ in range(NBUF)),
            tuple(pltpu.VMEM((K,), jnp.float32) for _ in range(NBUF)),
            pltpu.VMEM((NBUF, D), jnp.float32),
            pltpu.SemaphoreType.DMA,
            pltpu.SemaphoreType.DMA,
            tuple(pltpu.SemaphoreType.DMA for _ in range(NBUF)),
            tuple(pltpu.SemaphoreType.DMA for _ in range(NBUF)),
        ],
    )


def _sc_body(nodes_h, uuT_h, uul_h, w_h, out_h,
             idx_v, adjT, adj_v, na_v, rows, nbb, ostage,
             sem_a, sem_t, semr, semn):
    wid = lax.axis_index("s") * NC + lax.axis_index("c")
    base = wid * BPW

    pltpu.sync_copy(nodes_h.at[pl.ds(base, BPW)], idx_v)
    cn = pltpu.async_copy(uul_h.at[idx_v], na_v.at[pl.ds(0, BPW)], sem_a)
    cols = [pltpu.async_copy(uuT_h.at[k].at[idx_v], adjT.at[k], sem_t)
            for k in range(K)]

    lanes = lax.iota(jnp.int32, L)
    for k in range(K):
        cols[k].wait()
        for i in range(BPW // L):
            v = adjT[k, pl.ds(L * i, L)]
            plsc.store_scatter(adj_v, [lanes + (L * i), jnp.full((L,), k, jnp.int32)], v)
    cn.wait()

    def issue(j, b):
        pltpu.async_copy(w_h.at[adj_v.at[b]], rows[j], semr[j])
        pltpu.async_copy(uul_h.at[adj_v.at[b]], nbb[j], semn[j])

    for j in range(NBUF):
        issue(j, j)

    @pl.loop(0, BPW, step=NBUF)
    def _group(g):
        narv = _rsqrt(na_v[pl.ds(g, L)])
        for j in range(NBUF):
            b = g + j
            pltpu.make_async_copy(w_h.at[adj_v.at[b]], rows[j], semr[j]).wait()
            pltpu.make_async_copy(uul_h.at[adj_v.at[b]], nbb[j], semn[j]).wait()

            nar = jnp.broadcast_to(narv[j], (L,))
            wv = [_rsqrt(nbb[j][pl.ds(0, L)]) * nar,
                  _rsqrt(nbb[j][pl.ds(L, L)]) * nar]

            acc = [jnp.zeros((L,), jnp.float32) for _ in range(DB)]
            for k in range(K):
                wk = jnp.broadcast_to(wv[k // L][k % L], (L,))
                for dd in range(DB):
                    acc[dd] = acc[dd] + rows[j][k, pl.ds(L * dd, L)] * wk
            for dd in range(DB):
                ostage[j, pl.ds(L * dd, L)] = acc[dd]

            @pl.when(b + NBUF < BPW)
            def _refill():
                issue(j, b + NBUF)

        pltpu.sync_copy(ostage, out_h.at[pl.ds(base + g, NBUF)])


_sc_aggregate = _make_kernel()(_sc_body)


def kernel(nodes, u_u, u_u_l, u2e_weight):
    return _sc_aggregate(nodes, u_u.T, jnp.max(u_u_l, axis=1), u2e_weight)


# speedup vs baseline: 1.3149x; 1.3149x over previous
import functools

import jax
import jax.numpy as jnp
from jax import lax
from jax.experimental import pallas as pl
from jax.experimental.pallas import tpu as pltpu
from jax.experimental.pallas import tpu_sc as plsc

NC = 2
NS = 16
L = 16
NW = NC * NS

B = 4096
K = 32
D = 128
BPW = B // NW
DB = D // L
NBUF = 4


def _rsqrt(x):
    i = lax.bitcast_convert_type(x, jnp.int32)
    i = jnp.int32(0x5F3759DF) - jnp.right_shift(i, 1)
    y = lax.bitcast_convert_type(i, jnp.float32)
    for _ in range(3):
        y = y * (jnp.float32(1.5) - jnp.float32(0.5) * x * y * y)
    return y


_mesh = plsc.VectorSubcoreMesh(
    core_axis_name="c", subcore_axis_name="s", num_cores=NC, num_subcores=NS
)


def _make_kernel(interpret=False):
    return functools.partial(
        pl.kernel,
        out_type=jax.ShapeDtypeStruct((B, D), jnp.float32),
        mesh=_mesh,
        compiler_params=pltpu.CompilerParams(
            needs_layout_passes=False, use_tc_tiling_on_sc=False
        ),
        interpret=interpret,
        scratch_types=[
            pltpu.VMEM((BPW,), jnp.int32),
            pltpu.VMEM((K, BPW), jnp.int32),
            pltpu.VMEM((BPW, K), jnp.int32),
            pltpu.VMEM((BPW + L,), jnp.float32),
            tuple(pltpu.VMEM((K, D), jnp.float32) for _ in range(NBUF)),
            tuple(pltpu.VMEM((K,), jnp.float32) for _ in range(NBUF)),
            pltpu.VMEM((NBUF, D), jnp.float32),
            pltpu.SemaphoreType.DMA,
            pltpu.SemaphoreType.DMA,
            tuple(pltpu.SemaphoreType.DMA for _ in range(NBUF)),
            tuple(pltpu.SemaphoreType.DMA for _ in range(NBUF)),
        ],
    )


def _sc_body(nodes_h, uuT_h, uul_h, w_h, out_h,
             idx_v, adjT, adj_v, na_v, rows, nbb, ostage,
             sem_a, sem_t, semr, semn):
    wid = lax.axis_index("s") * NC + lax.axis_index("c")
    base = wid * BPW

    pltpu.sync_copy(nodes_h.at[pl.ds(base, BPW)], idx_v)
    cn = pltpu.async_copy(uul_h.at[idx_v], na_v.at[pl.ds(0, BPW)], sem_a)
    cols = [pltpu.async_copy(uuT_h.at[k].at[idx_v], adjT.at[k], sem_t)
            for k in range(K)]

    for c in cols:
        c.wait()
    lanes = lax.iota(jnp.int32, L)
    for k in range(K):
        for i in range(BPW // L):
            v = adjT[k, pl.ds(L * i, L)]
            plsc.store_scatter(adj_v, [lanes + (L * i), jnp.full((L,), k, jnp.int32)], v)
    cn.wait()

    def issue(j, b):
        pltpu.async_copy(w_h.at[adj_v.at[b]], rows[j], semr[j])
        pltpu.async_copy(uul_h.at[adj_v.at[b]], nbb[j], semn[j])

    for j in range(NBUF):
        issue(j, j)

    @pl.loop(0, BPW, step=NBUF)
    def _group(g):
        narv = _rsqrt(na_v[pl.ds(g, L)])
        for j in range(NBUF):
            b = g + j
            pltpu.make_async_copy(w_h.at[adj_v.at[b]], rows[j], semr[j]).wait()
            pltpu.make_async_copy(uul_h.at[adj_v.at[b]], nbb[j], semn[j]).wait()

            nar = jnp.broadcast_to(narv[j], (L,))
            wv = [_rsqrt(nbb[j][pl.ds(0, L)]) * nar,
                  _rsqrt(nbb[j][pl.ds(L, L)]) * nar]

            acc = [jnp.zeros((L,), jnp.float32) for _ in range(DB)]
            for k in range(K):
                wk = jnp.broadcast_to(wv[k // L][k % L], (L,))
                for dd in range(DB):
                    acc[dd] = acc[dd] + rows[j][k, pl.ds(L * dd, L)] * wk
            for dd in range(DB):
                ostage[j, pl.ds(L * dd, L)] = acc[dd]

            @pl.when(b + NBUF < BPW)
            def _refill():
                issue(j, b + NBUF)

        pltpu.sync_copy(ostage, out_h.at[pl.ds(base + g, NBUF)])


_sc_aggregate = _make_kernel()(_sc_body)


def kernel(nodes, u_u, u_u_l, u2e_weight):
    return _sc_aggregate(nodes, u_u.T, jnp.max(u_u_l, axis=1), u2e_weight)
